# Initial kernel scaffold; baseline (speedup 1.0000x reference)
import jax
import jax.numpy as jnp
from jax.experimental import pallas as pl


def kernel(xn, xe, edge_index, K1Nopen, K2Nopen, K1Eopen, K2Eopen, KNclose, alpha, KE1, KE2, KN1, KN2, lin1_w, lin1_b, lin2_w, lin2_b):
    raise NotImplementedError("write your pallas kernel here")


# speedup vs baseline: 13.4588x; 13.4588x over previous
import functools

import jax
import jax.numpy as jnp
from jax import lax
from jax.experimental import pallas as pl
from jax.experimental.pallas import tpu as pltpu
from jax.experimental.pallas import tpu_sc as plsc

N = 10000
E = 640000
H = 0.1
_INTERPRET = False


def _stats_matmul(xs, krons, rows_per_blk, offsets=None, rows=None):
    R = rows if rows is not None else xs[0].shape[0]
    if offsets is None:
        offsets = [0] * len(xs)
    Lout = krons[0].shape[1]
    nb = R // rows_per_blk
    assert R % rows_per_blk == 0

    def body(*refs):
        bi = pl.program_id(0)
        nx = len(xs)
        x_refs = refs[:nx]
        k_refs = refs[nx:2 * nx]
        h_ref, st_ref = refs[2 * nx], refs[2 * nx + 1]
        h = jnp.zeros((rows_per_blk, Lout), jnp.float32)
        for xr, kr in zip(x_refs, k_refs):
            h = h + jnp.dot(xr[...], kr[...], preferred_element_type=jnp.float32)
        h_ref[...] = h
        ps = jnp.sum(h, axis=0, keepdims=True)
        ps2 = jnp.sum(h * h, axis=0, keepdims=True)
        if Lout > 128:
            ps = ps.reshape(Lout // 128, 128).sum(axis=0, keepdims=True)
            ps2 = ps2.reshape(Lout // 128, 128).sum(axis=0, keepdims=True)

        @pl.when(bi == 0)
        def _():
            st_ref[...] = jnp.zeros((2, 128), jnp.float32)

        st_ref[0:1, :] += ps
        st_ref[1:2, :] += ps2

    in_specs = (
        [pl.BlockSpec((rows_per_blk, 128), functools.partial(lambda o, b: (b + o, 0), o))
         for o in offsets]
        + [pl.BlockSpec((128, Lout), lambda b: (0, 0)) for _ in krons]
    )
    out_specs = [
        pl.BlockSpec((rows_per_blk, Lout), lambda b: (b, 0)),
        pl.BlockSpec((2, 128), lambda b: (0, 0)),
    ]
    h, st = pl.pallas_call(
        body,
        grid=(nb,),
        in_specs=in_specs,
        out_specs=out_specs,
        out_shape=[
            jax.ShapeDtypeStruct((R, Lout), jnp.float32),
            jax.ShapeDtypeStruct((2, 128), jnp.float32),
        ],
        interpret=_INTERPRET,
    )(*xs, *krons)
    return h, st


def _apply_matmul(h, stats, kron2, count, rows_per_blk, resid=None, hscale=None):
    R, Lin = h.shape
    Lout = kron2.shape[1]
    nb = R // rows_per_blk
    assert R % rows_per_blk == 0

    def body(*refs):
        if resid is not None:
            h_ref, st_ref, k_ref, r_ref, o_ref = refs
        else:
            h_ref, st_ref, k_ref, o_ref = refs
            r_ref = None
        s = jnp.sum(st_ref[0, :])
        s2 = jnp.sum(st_ref[1, :])
        mean = s / count
        var = s2 / count - mean * mean
        inv = lax.rsqrt(var + 1e-5)
        g = jnp.tanh((h_ref[...] - mean) * inv)
        d = jnp.dot(g, k_ref[...], preferred_element_type=jnp.float32)
        if r_ref is not None:
            o_ref[...] = r_ref[...] + hscale * d
        else:
            o_ref[...] = d

    ins = [h, stats, kron2] + ([resid] if resid is not None else [])
    in_specs = [
        pl.BlockSpec((rows_per_blk, Lin), lambda b: (b, 0)),
        pl.BlockSpec((2, 128), lambda b: (0, 0)),
        pl.BlockSpec((Lin, Lout), lambda b: (0, 0)),
    ] + ([pl.BlockSpec((rows_per_blk, Lout), lambda b: (b, 0))] if resid is not None else [])
    out = pl.pallas_call(
        body,
        grid=(nb,),
        in_specs=in_specs,
        out_specs=pl.BlockSpec((rows_per_blk, Lout), lambda b: (b, 0)),
        out_shape=jax.ShapeDtypeStruct((R, Lout), jnp.float32),
        interpret=_INTERPRET,
    )(*ins)
    return out


def _node_double_layer(xs, krons, kron2, count, resid=None, hscale=None):
    Lout = kron2.shape[1]
    R = xs[0].shape[0]

    def body(*refs):
        nx = len(xs)
        x_refs = refs[:nx]
        k_refs = refs[nx:2 * nx]
        k2_ref = refs[2 * nx]
        if resid is not None:
            r_ref, o_ref = refs[2 * nx + 1], refs[2 * nx + 2]
        else:
            r_ref, o_ref = None, refs[2 * nx + 1]
        h = jnp.zeros((R, krons[0].shape[1]), jnp.float32)
        for xr, kr in zip(x_refs, k_refs):
            h = h + jnp.dot(xr[...], kr[...], preferred_element_type=jnp.float32)
        mean = jnp.sum(h) / count
        var = jnp.sum(h * h) / count - mean * mean
        g = jnp.tanh((h - mean) * lax.rsqrt(var + 1e-5))
        d = jnp.dot(g, k2_ref[...], preferred_element_type=jnp.float32)
        if r_ref is not None:
            o_ref[...] = r_ref[...] + hscale * d
        else:
            o_ref[...] = d

    ins = list(xs) + list(krons) + [kron2] + ([resid] if resid is not None else [])
    out = pl.pallas_call(
        body,
        out_shape=jax.ShapeDtypeStruct((R, Lout), jnp.float32),
        interpret=_INTERPRET,
    )(*ins)
    return out


def _close_mlp1(xn_r8, kron_close, kron_lin1, b1t):
    def body(x_ref, kc_ref, k1_ref, b_ref, o_ref):
        y = jnp.dot(x_ref[...], kc_ref[...], preferred_element_type=jnp.float32)
        t = jnp.dot(y, k1_ref[...], preferred_element_type=jnp.float32) + b_ref[...]
        o_ref[...] = jnp.where(t > 0, t, jnp.exp(jnp.minimum(t, 0.0)) - 1.0)

    return pl.pallas_call(
        body,
        out_shape=jax.ShapeDtypeStruct((xn_r8.shape[0], kron_lin1.shape[1]), jnp.float32),
        interpret=_INTERPRET,
    )(xn_r8, kron_close, kron_lin1, b1t)


def _close_mlp2(a, w2t, b2, rows_per_blk):
    R = a.shape[0]
    nb = R // rows_per_blk

    def body(a_ref, w_ref, b_ref, o_ref):
        z = jnp.dot(a_ref[...], w_ref[...], preferred_element_type=jnp.float32) + b_ref[...]
        m = jnp.max(z, axis=1, keepdims=True)
        lse = m + jnp.log(jnp.sum(jnp.exp(z - m), axis=1, keepdims=True))
        o_ref[...] = z - lse

    return pl.pallas_call(
        body,
        grid=(nb,),
        in_specs=[
            pl.BlockSpec((rows_per_blk, 256), lambda b: (b, 0)),
            pl.BlockSpec((256, 1024), lambda b: (0, 0)),
            pl.BlockSpec((1, 1024), lambda b: (0, 0)),
        ],
        out_specs=pl.BlockSpec((rows_per_blk, 1024), lambda b: (b, 0)),
        out_shape=jax.ShapeDtypeStruct((R, 1024), jnp.float32),
        interpret=_INTERPRET,
    )(a, w2t, b2)


_GATHER_WIN = 128
_SC_PARAMS = pltpu.CompilerParams(use_tc_tiling_on_sc=False)


def _sc_gather(table_rows, idx2e):
    n_idx = idx2e.shape[0]
    idx2e = idx2e.reshape(1, n_idx)
    mesh = plsc.VectorSubcoreMesh(core_axis_name="c", subcore_axis_name="s")

    @functools.partial(
        pl.kernel,
        out_type=jax.ShapeDtypeStruct((n_idx, 16), jnp.float32),
        mesh=mesh,
        compiler_params=_SC_PARAMS,
    )
    def k(x_hbm, i_hbm, o_hbm):
        def body(i_vmem, o_vmem):
            pltpu.sync_copy(x_hbm.at[i_vmem.at[0]], o_vmem)

        pltpu.emit_pipeline(
            body,
            grid=(n_idx // _GATHER_WIN,),
            in_specs=[pl.BlockSpec((1, _GATHER_WIN), lambda i: (0, i))],
            out_specs=[pl.BlockSpec((_GATHER_WIN, 16), lambda i: (i, 0))],
            core_axis_name=("c", "s"),
            dimension_semantics=(pltpu.PARALLEL,),
        )(i_hbm, o_hbm)

    return k(table_rows, idx2e)


def _sc_scatter(xe_rows, idx_i, idx_j, zeros_rows):
    mesh = plsc.VectorSubcoreMesh(core_axis_name="c", subcore_axis_name="s")
    per_tile = E // 32
    n_full = per_tile // _GATHER_WIN
    tail = per_tile - n_full * _GATHER_WIN

    @functools.partial(
        pl.kernel,
        out_type=jax.ShapeDtypeStruct((2, 2, N, 16), jnp.float32),
        mesh=mesh,
        scratch_types=[
            pltpu.VMEM_SHARED((N, 16), jnp.float32),
            pltpu.VMEM_SHARED((N, 16), jnp.float32),
            pltpu.VMEM((_GATHER_WIN,), jnp.int32),
            pltpu.VMEM((_GATHER_WIN, 16), jnp.float32),
            pltpu.VMEM((tail,), jnp.int32),
            pltpu.VMEM((tail, 16), jnp.float32),
        ],
        compiler_params=_SC_PARAMS,
    )
    def k(xe_hbm, i_hbm, j_hbm, z_hbm, o_hbm, acc_i, acc_j, ivm, rows, ivm_t, rows_t):
        c = lax.axis_index("c")
        s = lax.axis_index("s")

        @pl.when(s == 0)
        def _():
            pltpu.sync_copy(z_hbm, acc_i)
            pltpu.sync_copy(z_hbm, acc_j)

        plsc.subcore_barrier()

        base = (c * 16 + s) * per_tile

        def chunk(off, width, iv, rv):
            pltpu.sync_copy(xe_hbm.at[pl.ds(off, width)], rv)
            pltpu.sync_copy(i_hbm.at[pl.ds(off, width)], iv)
            pltpu.sync_copy(rv, acc_i.at[iv], add=True)
            pltpu.sync_copy(j_hbm.at[pl.ds(off, width)], iv)
            pltpu.sync_copy(rv, acc_j.at[iv], add=True)

        @pl.loop(0, n_full)
        def _(t):
            chunk(base + t * _GATHER_WIN, _GATHER_WIN, ivm, rows)

        chunk(base + n_full * _GATHER_WIN, tail, ivm_t, rows_t)

        plsc.subcore_barrier()

        @pl.when(s == 0)
        def _():
            pltpu.sync_copy(acc_i, o_hbm.at[c, 0])
            pltpu.sync_copy(acc_j, o_hbm.at[c, 1])

    return k(xe_rows, idx_i, idx_j, zeros_rows)


def _kron8(w):
    return jnp.kron(jnp.eye(8, dtype=jnp.float32), w.T)


def _kron16(w):
    return jnp.kron(jnp.eye(16, dtype=jnp.float32), w.T)


def _pad8(w):
    return jnp.pad(w, ((0, 0), (0, 8 - w.shape[1])))


def _rows_from_bcn(x):
    m = x.shape[2]
    xt = jnp.transpose(x[0])
    return jnp.pad(xt, ((0, 0), (0, 8 - xt.shape[1])))


def kernel(xn, xe, edge_index, K1Nopen, K2Nopen, K1Eopen, K2Eopen, KNclose,
           alpha, KE1, KE2, KN1, KN2, lin1_w, lin1_b, lin2_w, lin2_b):
    f32 = jnp.float32
    idx_i = edge_index[0]
    idx_j = edge_index[1]
    idx2e = jnp.concatenate([idx_i, idx_j])

    kn1o = _kron16(_pad8(K1Nopen))
    kn2o = _kron16(K2Nopen)
    ke1o = _kron16(_pad8(K1Eopen))
    ke2o = _kron8(K2Eopen)
    kA, kB, kC, k2e = [], [], [], []
    kU, kV, kR, k2n = [], [], [], []
    for i in range(KE1.shape[0]):
        P, C, G = KE1[i][:, 0:16], KE1[i][:, 16:32], KE1[i][:, 32:48]
        kA.append(_kron8(P / 2 + G))
        kB.append(_kron8(P / 2 - G))
        kC.append(_kron8(C))
        k2e.append(_kron8(KE2[i]))
        Pn, Qn, Rn = KN1[i][:, 0:16], KN1[i][:, 16:32], KN1[i][:, 32:48]
        kU.append(_kron8(Pn / 2 + Qn))
        kV.append(_kron8(Pn / 2 - Qn))
        kR.append(_kron8(Rn))
        k2n.append(_kron8(KN2[i]))
    kron_close = _kron8(KNclose)
    kron_lin1 = jnp.kron(jnp.eye(8, dtype=f32), lin1_w.T)
    b1t = jnp.tile(lin1_b, 8).reshape(1, 2048)
    w2t = lin2_w.T
    b2 = lin2_b.reshape(1, 1024)
    zeros_rows = jnp.zeros((N, 16), f32)

    xn8 = _rows_from_bcn(xn).reshape(N // 16, 128)
    xe8 = _rows_from_bcn(xe).reshape(E // 16, 128)

    xn_r8 = _node_double_layer([xn8], [kn1o], kn2o, float(16 * N)).reshape(N // 8, 128)

    h, st = _stats_matmul([xe8], [ke1o], rows_per_blk=4000)
    h = h.reshape(E // 8, 128)
    xe_r8 = _apply_matmul(h, st, ke2o, float(16 * E), rows_per_blk=8000)

    for i in range(KE1.shape[0]):
        xij = _sc_gather(xn_r8.reshape(N, 16), idx2e).reshape(2 * E // 8, 128)
        h, st = _stats_matmul([xij, xij, xe_r8], [kA[i], kB[i], kC[i]],
                              rows_per_blk=8000,
                              offsets=[0, (E // 8) // 8000, 0], rows=E // 8)
        xe_r8 = _apply_matmul(h, st, k2e[i], float(16 * E), rows_per_blk=8000,
                              resid=xe_r8, hscale=H)
        P = _sc_scatter(xe_r8.reshape(E, 16), idx_i, idx_j, zeros_rows)
        Pr = P.reshape(4, N // 8, 128)
        si = Pr[0] + Pr[2]
        sj = Pr[1] + Pr[3]
        xn_r8 = _node_double_layer([si, sj, xn_r8], [kU[i], kV[i], kR[i]],
                                   k2n[i], float(16 * N), resid=xn_r8, hscale=H)

    a = _close_mlp1(xn_r8, kron_close, kron_lin1, b1t)
    out = _close_mlp2(a.reshape(N, 256), w2t, b2, rows_per_blk=1000)
    return (out, jax.nn.sigmoid(alpha))


# speedup vs baseline: 21.9114x; 1.6280x over previous
import functools

import jax
import jax.numpy as jnp
from jax import lax
from jax.experimental import pallas as pl
from jax.experimental.pallas import tpu as pltpu
from jax.experimental.pallas import tpu_sc as plsc

N = 10000
E = 640000
H = 0.1
_INTERPRET = False


def _stats_matmul(xs, krons, rows_per_blk, offsets=None, rows=None):
    R = rows if rows is not None else xs[0].shape[0]
    if offsets is None:
        offsets = [0] * len(xs)
    Lout = krons[0].shape[1]
    nb = R // rows_per_blk
    assert R % rows_per_blk == 0

    def body(*refs):
        bi = pl.program_id(0)
        nx = len(xs)
        x_refs = refs[:nx]
        k_refs = refs[nx:2 * nx]
        h_ref, st_ref = refs[2 * nx], refs[2 * nx + 1]
        h = jnp.zeros((rows_per_blk, Lout), jnp.float32)
        for xr, kr in zip(x_refs, k_refs):
            h = h + jnp.dot(xr[...], kr[...], preferred_element_type=jnp.float32)
        h_ref[...] = h
        ps = jnp.sum(h, axis=0, keepdims=True)
        ps2 = jnp.sum(h * h, axis=0, keepdims=True)
        if Lout > 128:
            ps = ps.reshape(Lout // 128, 128).sum(axis=0, keepdims=True)
            ps2 = ps2.reshape(Lout // 128, 128).sum(axis=0, keepdims=True)

        @pl.when(bi == 0)
        def _():
            st_ref[...] = jnp.zeros((2, 128), jnp.float32)

        st_ref[0:1, :] += ps
        st_ref[1:2, :] += ps2

    in_specs = (
        [pl.BlockSpec((rows_per_blk, 128), functools.partial(lambda o, b: (b + o, 0), o))
         for o in offsets]
        + [pl.BlockSpec((128, Lout), lambda b: (0, 0)) for _ in krons]
    )
    out_specs = [
        pl.BlockSpec((rows_per_blk, Lout), lambda b: (b, 0)),
        pl.BlockSpec((2, 128), lambda b: (0, 0)),
    ]
    h, st = pl.pallas_call(
        body,
        grid=(nb,),
        in_specs=in_specs,
        out_specs=out_specs,
        out_shape=[
            jax.ShapeDtypeStruct((R, Lout), jnp.float32),
            jax.ShapeDtypeStruct((2, 128), jnp.float32),
        ],
        interpret=_INTERPRET,
    )(*xs, *krons)
    return h, st


def _apply_matmul(h, stats, kron2, count, rows_per_blk, resid=None, hscale=None):
    R, Lin = h.shape
    Lout = kron2.shape[1]
    nb = R // rows_per_blk
    assert R % rows_per_blk == 0

    def body(*refs):
        if resid is not None:
            h_ref, st_ref, k_ref, r_ref, o_ref = refs
        else:
            h_ref, st_ref, k_ref, o_ref = refs
            r_ref = None
        s = jnp.sum(st_ref[0, :])
        s2 = jnp.sum(st_ref[1, :])
        mean = s / count
        var = s2 / count - mean * mean
        inv = lax.rsqrt(var + 1e-5)
        g = jnp.tanh((h_ref[...] - mean) * inv)
        d = jnp.dot(g, k_ref[...], preferred_element_type=jnp.float32)
        if r_ref is not None:
            o_ref[...] = r_ref[...] + hscale * d
        else:
            o_ref[...] = d

    ins = [h, stats, kron2] + ([resid] if resid is not None else [])
    in_specs = [
        pl.BlockSpec((rows_per_blk, Lin), lambda b: (b, 0)),
        pl.BlockSpec((2, 128), lambda b: (0, 0)),
        pl.BlockSpec((Lin, Lout), lambda b: (0, 0)),
    ] + ([pl.BlockSpec((rows_per_blk, Lout), lambda b: (b, 0))] if resid is not None else [])
    out = pl.pallas_call(
        body,
        grid=(nb,),
        in_specs=in_specs,
        out_specs=pl.BlockSpec((rows_per_blk, Lout), lambda b: (b, 0)),
        out_shape=jax.ShapeDtypeStruct((R, Lout), jnp.float32),
        interpret=_INTERPRET,
    )(*ins)
    return out


def _open_stats(x_b3m, w1, blk_m):
    M = x_b3m.shape[2]
    nb = M // blk_m
    assert M % blk_m == 0

    def body(x_ref, w_ref, h_ref, st_ref, acc_ref):
        bi = pl.program_id(0)
        h = lax.dot_general(w_ref[...], x_ref[0],
                            (((1,), (0,)), ((), ())),
                            preferred_element_type=jnp.float32)
        h_ref[...] = h

        @pl.when(bi == 0)
        def _():
            acc_ref[0] = 0.0
            acc_ref[1] = 0.0

        acc_ref[0] += jnp.sum(h)
        acc_ref[1] += jnp.sum(h * h)

        @pl.when(bi == nb - 1)
        def _():
            o = jnp.ones((1, 128), jnp.float32)
            st_ref[0:1, :] = o * (acc_ref[0] / 128.0)
            st_ref[1:2, :] = o * (acc_ref[1] / 128.0)

    h, st = pl.pallas_call(
        body,
        grid=(nb,),
        in_specs=[
            pl.BlockSpec((1, 3, blk_m), lambda b: (0, 0, b)),
            pl.BlockSpec((16, 3), lambda b: (0, 0)),
        ],
        out_specs=[
            pl.BlockSpec((16, blk_m), lambda b: (0, b)),
            pl.BlockSpec((2, 128), lambda b: (0, 0)),
        ],
        out_shape=[
            jax.ShapeDtypeStruct((16, M), jnp.float32),
            jax.ShapeDtypeStruct((2, 128), jnp.float32),
        ],
        scratch_shapes=[pltpu.SMEM((2,), jnp.float32)],
        interpret=_INTERPRET,
    )(x_b3m, w1)
    return h, st


def _open_apply(h_cm, stats, w2, count, blk_m):
    M = h_cm.shape[1]
    nb = M // blk_m

    def body(h_ref, st_ref, w_ref, o_ref):
        s = jnp.sum(st_ref[0, :])
        s2 = jnp.sum(st_ref[1, :])
        mean = s / count
        inv = lax.rsqrt(s2 / count - mean * mean + 1e-5)
        g = jnp.tanh((h_ref[...] - mean) * inv)
        o_ref[...] = lax.dot_general(g, w_ref[...], (((0,), (1,)), ((), ())),
                                     preferred_element_type=jnp.float32)

    return pl.pallas_call(
        body,
        grid=(nb,),
        in_specs=[
            pl.BlockSpec((16, blk_m), lambda b: (0, b)),
            pl.BlockSpec((2, 128), lambda b: (0, 0)),
            pl.BlockSpec((16, 16), lambda b: (0, 0)),
        ],
        out_specs=pl.BlockSpec((blk_m, 16), lambda b: (b, 0)),
        out_shape=jax.ShapeDtypeStruct((M, 16), jnp.float32),
        interpret=_INTERPRET,
    )(h_cm, stats, w2)


def _node_double_layer(xs, krons, kron2, count, resid=None, hscale=None):
    Lout = kron2.shape[1]
    R = xs[0].shape[0]

    def body(*refs):
        nx = len(xs)
        x_refs = refs[:nx]
        k_refs = refs[nx:2 * nx]
        k2_ref = refs[2 * nx]
        if resid is not None:
            r_ref, o_ref = refs[2 * nx + 1], refs[2 * nx + 2]
        else:
            r_ref, o_ref = None, refs[2 * nx + 1]
        h = jnp.zeros((R, krons[0].shape[1]), jnp.float32)
        for xr, kr in zip(x_refs, k_refs):
            h = h + jnp.dot(xr[...], kr[...], preferred_element_type=jnp.float32)
        mean = jnp.sum(h) / count
        var = jnp.sum(h * h) / count - mean * mean
        g = jnp.tanh((h - mean) * lax.rsqrt(var + 1e-5))
        d = jnp.dot(g, k2_ref[...], preferred_element_type=jnp.float32)
        if r_ref is not None:
            o_ref[...] = r_ref[...] + hscale * d
        else:
            o_ref[...] = d

    ins = list(xs) + list(krons) + [kron2] + ([resid] if resid is not None else [])
    out = pl.pallas_call(
        body,
        out_shape=jax.ShapeDtypeStruct((R, Lout), jnp.float32),
        interpret=_INTERPRET,
    )(*ins)
    return out


def _close_mlp1(xn_r8, kron_close, kron_lin1, b1t):
    def body(x_ref, kc_ref, k1_ref, b_ref, o_ref):
        y = jnp.dot(x_ref[...], kc_ref[...], preferred_element_type=jnp.float32)
        t = jnp.dot(y, k1_ref[...], preferred_element_type=jnp.float32) + b_ref[...]
        o_ref[...] = jnp.where(t > 0, t, jnp.exp(jnp.minimum(t, 0.0)) - 1.0)

    return pl.pallas_call(
        body,
        out_shape=jax.ShapeDtypeStruct((xn_r8.shape[0], kron_lin1.shape[1]), jnp.float32),
        interpret=_INTERPRET,
    )(xn_r8, kron_close, kron_lin1, b1t)


def _close_mlp2(a, w2t, b2, rows_per_blk):
    R = a.shape[0]
    nb = R // rows_per_blk

    def body(a_ref, w_ref, b_ref, o_ref):
        z = jnp.dot(a_ref[...], w_ref[...], preferred_element_type=jnp.float32) + b_ref[...]
        m = jnp.max(z, axis=1, keepdims=True)
        lse = m + jnp.log(jnp.sum(jnp.exp(z - m), axis=1, keepdims=True))
        o_ref[...] = z - lse

    return pl.pallas_call(
        body,
        grid=(nb,),
        in_specs=[
            pl.BlockSpec((rows_per_blk, 256), lambda b: (b, 0)),
            pl.BlockSpec((256, 1024), lambda b: (0, 0)),
            pl.BlockSpec((1, 1024), lambda b: (0, 0)),
        ],
        out_specs=pl.BlockSpec((rows_per_blk, 1024), lambda b: (b, 0)),
        out_shape=jax.ShapeDtypeStruct((R, 1024), jnp.float32),
        interpret=_INTERPRET,
    )(a, w2t, b2)


_GATHER_WIN = 128
_SC_PARAMS = pltpu.CompilerParams(use_tc_tiling_on_sc=False)


def _sc_gather(table_rows, idx2e):
    n_idx = idx2e.shape[0]
    idx2e = idx2e.reshape(1, n_idx)
    mesh = plsc.VectorSubcoreMesh(core_axis_name="c", subcore_axis_name="s")

    @functools.partial(
        pl.kernel,
        out_type=jax.ShapeDtypeStruct((n_idx, 16), jnp.float32),
        mesh=mesh,
        compiler_params=_SC_PARAMS,
    )
    def k(x_hbm, i_hbm, o_hbm):
        def body(i_vmem, o_vmem):
            pltpu.sync_copy(x_hbm.at[i_vmem.at[0]], o_vmem)

        pltpu.emit_pipeline(
            body,
            grid=(n_idx // _GATHER_WIN,),
            in_specs=[pl.BlockSpec((1, _GATHER_WIN), lambda i: (0, i))],
            out_specs=[pl.BlockSpec((_GATHER_WIN, 16), lambda i: (i, 0))],
            core_axis_name=("c", "s"),
            dimension_semantics=(pltpu.PARALLEL,),
        )(i_hbm, o_hbm)

    return k(table_rows, idx2e)


def _sc_scatter(xe_rows, edge_index, zeros_rows):
    mesh = plsc.VectorSubcoreMesh(core_axis_name="c", subcore_axis_name="s")
    per_tile = E // 32
    n_full = per_tile // _GATHER_WIN
    tail = per_tile - n_full * _GATHER_WIN

    @functools.partial(
        pl.kernel,
        out_type=jax.ShapeDtypeStruct((2, 2, N, 16), jnp.float32),
        mesh=mesh,
        scratch_types=[
            pltpu.VMEM_SHARED((N, 16), jnp.float32),
            pltpu.VMEM_SHARED((N, 16), jnp.float32),
            pltpu.VMEM((_GATHER_WIN,), jnp.int32),
            pltpu.VMEM((_GATHER_WIN, 16), jnp.float32),
            pltpu.VMEM((tail,), jnp.int32),
            pltpu.VMEM((tail, 16), jnp.float32),
        ],
        compiler_params=_SC_PARAMS,
    )
    def k(xe_hbm, ij_hbm, z_hbm, o_hbm, acc_i, acc_j, ivm, rows, ivm_t, rows_t):
        c = lax.axis_index("c")
        s = lax.axis_index("s")

        @pl.when(s == 0)
        def _():
            pltpu.sync_copy(z_hbm, acc_i)
            pltpu.sync_copy(z_hbm, acc_j)

        plsc.subcore_barrier()

        base = (c * 16 + s) * per_tile

        def chunk(off, width, iv, rv):
            pltpu.sync_copy(xe_hbm.at[pl.ds(off, width)], rv)
            pltpu.sync_copy(ij_hbm.at[0, pl.ds(off, width)], iv)
            pltpu.sync_copy(rv, acc_i.at[iv], add=True)
            pltpu.sync_copy(ij_hbm.at[1, pl.ds(off, width)], iv)
            pltpu.sync_copy(rv, acc_j.at[iv], add=True)

        @pl.loop(0, n_full)
        def _(t):
            chunk(base + t * _GATHER_WIN, _GATHER_WIN, ivm, rows)

        chunk(base + n_full * _GATHER_WIN, tail, ivm_t, rows_t)

        plsc.subcore_barrier()

        @pl.when(s == 0)
        def _():
            pltpu.sync_copy(acc_i, o_hbm.at[c, 0])
            pltpu.sync_copy(acc_j, o_hbm.at[c, 1])

    return k(xe_rows, edge_index, zeros_rows)


def _kron8(w):
    return jnp.kron(jnp.eye(8, dtype=jnp.float32), w.T)


def kernel(xn, xe, edge_index, K1Nopen, K2Nopen, K1Eopen, K2Eopen, KNclose,
           alpha, KE1, KE2, KN1, KN2, lin1_w, lin1_b, lin2_w, lin2_b):
    f32 = jnp.float32
    idx2e = edge_index.reshape(2 * E)

    kA, kB, kC, k2e = [], [], [], []
    kU, kV, kR, k2n = [], [], [], []
    for i in range(KE1.shape[0]):
        P, C, G = KE1[i][:, 0:16], KE1[i][:, 16:32], KE1[i][:, 32:48]
        kA.append(_kron8(P / 2 + G))
        kB.append(_kron8(P / 2 - G))
        kC.append(_kron8(C))
        k2e.append(_kron8(KE2[i]))
        Pn, Qn, Rn = KN1[i][:, 0:16], KN1[i][:, 16:32], KN1[i][:, 32:48]
        kU.append(_kron8(Pn / 2 + Qn))
        kV.append(_kron8(Pn / 2 - Qn))
        kR.append(_kron8(Rn))
        k2n.append(_kron8(KN2[i]))
    kron_close = _kron8(KNclose)
    kron_lin1 = jnp.kron(jnp.eye(8, dtype=f32), lin1_w.T)
    b1t = jnp.tile(lin1_b, 8).reshape(1, 2048)
    w2t = lin2_w.T
    b2 = lin2_b.reshape(1, 1024)
    zeros_rows = jnp.zeros((N, 16), f32)

    hn, stn = _open_stats(xn, K1Nopen, blk_m=N)
    xn_r8 = _open_apply(hn, stn, K2Nopen, float(16 * N), blk_m=N).reshape(N // 8, 128)

    he, ste = _open_stats(xe, K1Eopen, blk_m=6400)
    xe_r8 = _open_apply(he, ste, K2Eopen, float(16 * E), blk_m=6400).reshape(E // 8, 128)

    for i in range(KE1.shape[0]):
        xij = _sc_gather(xn_r8.reshape(N, 16), idx2e).reshape(2 * E // 8, 128)
        h, st = _stats_matmul([xij, xij, xe_r8], [kA[i], kB[i], kC[i]],
                              rows_per_blk=8000,
                              offsets=[0, (E // 8) // 8000, 0], rows=E // 8)
        xe_r8 = _apply_matmul(h, st, k2e[i], float(16 * E), rows_per_blk=8000,
                              resid=xe_r8, hscale=H)
        P = _sc_scatter(xe_r8.reshape(E, 16), edge_index, zeros_rows)
        Pr = P.reshape(4, N // 8, 128)
        si = Pr[0] + Pr[2]
        sj = Pr[1] + Pr[3]
        xn_r8 = _node_double_layer([si, sj, xn_r8], [kU[i], kV[i], kR[i]],
                                   k2n[i], float(16 * N), resid=xn_r8, hscale=H)

    a = _close_mlp1(xn_r8, kron_close, kron_lin1, b1t)
    out = _close_mlp2(a.reshape(N, 256), w2t, b2, rows_per_blk=1000)
    return (out, jax.nn.sigmoid(alpha))
